# baseline (speedup 1.0000x reference)
import functools

import jax
import jax.numpy as jnp
from jax import lax
from jax.experimental import pallas as pl
from jax.experimental.pallas import tpu as pltpu
from jax.experimental.pallas import tpu_sc as plsc

N = 50000
E = 800000
NG = 512
H = 64
HH = 32

NC = 2
NS = 16
SLAB = 3136
NPAD = NS * SLAB
BN = SLAB
GRID = NPAD // BN
PK = NPAD * HH // 128
PB = PK // GRID

F32 = jnp.float32


@functools.cache
def _mesh():
    return plsc.VectorSubcoreMesh(core_axis_name="c", subcore_axis_name="s",
                                  num_cores=NC, num_subcores=NS)


EC = E // (NC * NS)
DC = 1000
AC0 = 200
AC32 = 400
EPS = E // NS
SG = 5


def _fill_f32(ref, n, val):
    def body(i, _):
        ref[pl.ds(i * 16, 16)] = jnp.full((16,), val, F32)
        return 0
    lax.fori_loop(0, n // 16, body, 0)


def _fill_rows32(ref, nrows):
    def body(i, _):
        ref[i, pl.ds(0, 16)] = jnp.zeros((16,), F32)
        ref[i, pl.ds(16, 16)] = jnp.zeros((16,), F32)
        return 0
    lax.fori_loop(0, nrows, body, 0)


def _deg_body(dst_hbm, out0, out1, dst_v, ones_v, zrow_v, acc, sem):
    c = lax.axis_index("c")
    s = lax.axis_index("s")
    _fill_f32(zrow_v, SLAB, 0.0)
    _fill_f32(ones_v, DC, 1.0)
    pltpu.sync_copy(zrow_v, acc.at[pl.ds(s * SLAB, SLAB)])
    plsc.subcore_barrier()
    wid = s * NC + c
    def chunk(k, _):
        base = wid * EC + k * DC
        pltpu.sync_copy(dst_hbm.at[pl.ds(base, DC)], dst_v)
        pltpu.async_copy(ones_v, acc.at[dst_v], sem, add=True).wait()
        return 0
    lax.fori_loop(0, EC // DC, chunk, 0)
    plsc.subcore_barrier()
    sl = pl.ds(s * SLAB, SLAB)
    pltpu.sync_copy(acc.at[sl], zrow_v)
    @pl.when(c == 0)
    def _():
        pltpu.sync_copy(zrow_v, out0.at[sl])
    @pl.when(c == 1)
    def _():
        pltpu.sync_copy(zrow_v, out1.at[sl])


@functools.cache
def _deg_call():
    return pl.kernel(
        _deg_body,
        out_type=(jax.ShapeDtypeStruct((NPAD,), F32),
                  jax.ShapeDtypeStruct((NPAD,), F32)),
        mesh=_mesh(),
        scratch_types=[
            pltpu.VMEM((DC,), jnp.int32),
            pltpu.VMEM((DC,), F32),
            pltpu.VMEM((SLAB,), F32),
            pltpu.VMEM_SHARED((NPAD,), F32),
            pltpu.SemaphoreType.DMA,
        ],
        compiler_params=pltpu.CompilerParams(use_tc_tiling_on_sc=False),
    )


def _agg_pipeline(tab_hbm, src2_hbm, dst2_hbm, acc, row0, nch, ac,
                  srcb, dstb, rows, sems):
    nsg = nch // SG
    pltpu.sync_copy(src2_hbm.at[pl.ds(row0, SG)], srcb)
    pltpu.sync_copy(dst2_hbm.at[pl.ds(row0, SG)], dstb)
    pltpu.async_copy(tab_hbm.at[srcb.at[0]], rows[0], sems[0])

    def group(sg, _):
        for j in range(SG - 1):
            p = j % 2
            pltpu.async_copy(tab_hbm.at[srcb.at[j + 1]], rows[1 - p],
                             sems[1 - p])
            pltpu.make_async_copy(tab_hbm.at[srcb.at[j]], rows[p],
                                  sems[p]).wait()
            pltpu.sync_copy(rows[p], acc.at[dstb.at[j]], add=True)
        p = (SG - 1) % 2
        pltpu.make_async_copy(tab_hbm.at[srcb.at[SG - 1]], rows[p],
                              sems[p]).wait()
        pltpu.sync_copy(rows[p], acc.at[dstb.at[SG - 1]], add=True)

        @pl.when(sg < nsg - 1)
        def _():
            nrow = row0 + (sg + 1) * SG
            pltpu.sync_copy(src2_hbm.at[pl.ds(nrow, SG)], srcb)
            pltpu.sync_copy(dst2_hbm.at[pl.ds(nrow, SG)], dstb)
            pltpu.async_copy(tab_hbm.at[srcb.at[0]], rows[0], sems[0])
        return 0

    lax.fori_loop(0, nsg, group, 0)


def _agg_scratch(ac):
    return (
        [pltpu.VMEM((SG, ac), jnp.int32)] * 2
        + [pltpu.VMEM((ac, HH), F32)] * 2
        + [pltpu.VMEM_SHARED((NPAD, HH), F32)]
        + [pltpu.SemaphoreType.DMA] * 2
    )


def _stg_rows(ac):
    return 392 if ac >= 392 else 196


def _zero_acc(acc, s, rows0, stgn):
    _fill_rows32(rows0, stgn)
    stg = rows0.at[pl.ds(0, stgn)]
    for j in range(SLAB // stgn):
        pltpu.sync_copy(stg, acc.at[pl.ds(s * SLAB + j * stgn, stgn)])


def _acc_to_out(acc, s, rows0, out, stgn):
    stg = rows0.at[pl.ds(0, stgn)]
    for j in range(SLAB // stgn):
        sl = pl.ds(s * SLAB + j * stgn, stgn)
        pltpu.sync_copy(acc.at[sl], stg)
        pltpu.sync_copy(stg, out.at[sl])


def _aggl0_body(tab_hbm, srcd_hbm, dstd_hbm, out0, out1, *sc):
    srcb, dstb, rows, acc, sems = sc[0], sc[1], sc[2:4], sc[4], sc[5:]
    c = lax.axis_index("c")
    s = lax.axis_index("s")
    _zero_acc(acc, s, rows[0], _stg_rows(AC0))
    plsc.subcore_barrier()
    wid = s * NC + c
    _agg_pipeline(tab_hbm, srcd_hbm, dstd_hbm, acc, wid * (EC // AC0),
                  EC // AC0, AC0, srcb, dstb, rows, sems)
    plsc.subcore_barrier()
    @pl.when(c == 0)
    def _():
        _acc_to_out(acc, s, rows[0], out0, _stg_rows(AC0))
    @pl.when(c == 1)
    def _():
        _acc_to_out(acc, s, rows[0], out1, _stg_rows(AC0))


@functools.cache
def _aggl0_call():
    return pl.kernel(
        _aggl0_body,
        out_type=(jax.ShapeDtypeStruct((NPAD, HH), F32),
                  jax.ShapeDtypeStruct((NPAD, HH), F32)),
        mesh=_mesh(),
        scratch_types=_agg_scratch(AC0),
        compiler_params=pltpu.CompilerParams(use_tc_tiling_on_sc=False),
    )


def _agg32_body(taba_hbm, tabb_hbm, srcd_hbm, dstd_hbm, outa, outb, *sc):
    srcb, dstb, rows, acc, sems = sc[0], sc[1], sc[2:4], sc[4], sc[5:]
    c = lax.axis_index("c")
    s = lax.axis_index("s")

    def half(tab_hbm, out):
        _zero_acc(acc, s, rows[0], _stg_rows(AC32))
        plsc.subcore_barrier()
        _agg_pipeline(tab_hbm, srcd_hbm, dstd_hbm, acc, s * (EPS // AC32),
                      EPS // AC32, AC32, srcb, dstb, rows, sems)
        plsc.subcore_barrier()
        _acc_to_out(acc, s, rows[0], out, _stg_rows(AC32))

    @pl.when(c == 0)
    def _():
        half(taba_hbm, outa)
    @pl.when(c == 1)
    def _():
        half(tabb_hbm, outb)


@functools.cache
def _agg32_call():
    return pl.kernel(
        _agg32_body,
        out_type=(jax.ShapeDtypeStruct((NPAD, HH), F32),
                  jax.ShapeDtypeStruct((NPAD, HH), F32)),
        mesh=_mesh(),
        scratch_types=_agg_scratch(AC32),
        compiler_params=pltpu.CompilerParams(use_tc_tiling_on_sc=False),
    )


def _prow(w=128):
    return pl.BlockSpec((PB, w), lambda i: (i, 0))


def _full_spec(a, b):
    return pl.BlockSpec((a, b), lambda i: (0, 0))


def _prep_body(x4_ref, dg_ref, sel_ref, dinv_ref, gx_ref):
    r = lax.rsqrt(dg_ref[...] + 1.0)
    dv = jnp.concatenate(
        [jnp.broadcast_to(r[:, j:j + 1], (PB, HH)) for j in range(4)], axis=1)
    dinv_ref[...] = dv
    gx_ref[...] = _mm(x4_ref[...], sel_ref[...]) * dv


def _prep_call(x4, dg, sel):
    return pl.pallas_call(
        _prep_body,
        grid=(GRID,),
        in_specs=[_prow(8), _prow(4), _full_spec(8, 128)],
        out_specs=(_prow(), _prow()),
        out_shape=(jax.ShapeDtypeStruct((PK, 128), F32),
                   jax.ShapeDtypeStruct((PK, 128), F32)),
    )(x4, dg, sel)


def _mm(a, b):
    return jnp.dot(a, b, preferred_element_type=F32)


def _l1_body(t0a, t0b, gx, dinv, w1a, w1b, b1a, b1b,
             w2aa, w2ab, w2ba, w2bb, ga_ref, gb_ref):
    d = dinv[...]
    q = (t0a[...] + t0b[...] + gx[...]) * d
    y1a = jnp.maximum(_mm(q, w1a[...]) + b1a[...], 0.0)
    y1b = jnp.maximum(_mm(q, w1b[...]) + b1b[...], 0.0)
    ga_ref[...] = (_mm(y1a, w2aa[...]) + _mm(y1b, w2ba[...])) * d
    gb_ref[...] = (_mm(y1a, w2ab[...]) + _mm(y1b, w2bb[...])) * d


def _l1_call(t0a, t0b, gx, dinv, w1a, w1b, b1a, b1b, w2aa, w2ab, w2ba, w2bb):
    return pl.pallas_call(
        _l1_body,
        grid=(GRID,),
        in_specs=[_prow()] * 4 + [_full_spec(128, 128)] * 2
        + [_full_spec(1, 128)] * 2 + [_full_spec(128, 128)] * 4,
        out_specs=(_prow(), _prow()),
        out_shape=(jax.ShapeDtypeStruct((PK, 128), F32),
                   jax.ShapeDtypeStruct((PK, 128), F32)),
    )(t0a, t0b, gx, dinv, w1a, w1b, b1a, b1b, w2aa, w2ab, w2ba, w2bb)


def _l2_body(ta, tb, ga, gb, dinv, b2a, b2b,
             w3aa, w3ab, w3ba, w3bb, oa_ref, ob_ref):
    d = dinv[...]
    u0 = jnp.maximum((ta[...] + ga[...]) * d + b2a[...], 0.0)
    u1 = jnp.maximum((tb[...] + gb[...]) * d + b2b[...], 0.0)
    oa_ref[...] = (_mm(u0, w3aa[...]) + _mm(u1, w3ba[...])) * d
    ob_ref[...] = (_mm(u0, w3ab[...]) + _mm(u1, w3bb[...])) * d


def _l2_call(ta, tb, ga, gb, dinv, b2a, b2b, w3aa, w3ab, w3ba, w3bb):
    return pl.pallas_call(
        _l2_body,
        grid=(GRID,),
        in_specs=[_prow()] * 5 + [_full_spec(1, 128)] * 2
        + [_full_spec(128, 128)] * 4,
        out_specs=(_prow(), _prow()),
        out_shape=(jax.ShapeDtypeStruct((PK, 128), F32),
                   jax.ShapeDtypeStruct((PK, 128), F32)),
    )(ta, tb, ga, gb, dinv, b2a, b2b, w3aa, w3ab, w3ba, w3bb)


def _pool_body(ta, tb, ga, gb, dinv, b3a, b3b, batch,
               s0_ref, s1_ref, cnt_ref):
    i = pl.program_id(0)
    d = dinv[...]
    v0 = jnp.maximum((ta[...] + ga[...]) * d + b3a[...], 0.0)
    v1 = jnp.maximum((tb[...] + gb[...]) * d + b3b[...], 0.0)
    ids = batch[...]
    seg = lax.broadcasted_iota(jnp.int32, (PB, NG), 1)
    dn = (((0,), (0,)), ((), ()))
    v0h = v0.astype(jnp.bfloat16)
    v1h = v1.astype(jnp.bfloat16)
    ones = jnp.ones((PB, 1), jnp.bfloat16)
    s0 = jnp.zeros((NG, HH), F32)
    s1 = jnp.zeros((NG, HH), F32)
    cnt = jnp.zeros((NG, 1), F32)
    for j in range(4):
        p = (ids[:, j:j + 1] == seg).astype(jnp.bfloat16)
        s0 = s0 + lax.dot_general(p, v0h[:, HH * j:HH * (j + 1)], dn,
                                  preferred_element_type=F32)
        s1 = s1 + lax.dot_general(p, v1h[:, HH * j:HH * (j + 1)], dn,
                                  preferred_element_type=F32)
        cnt = cnt + lax.dot_general(p, ones, dn,
                                    preferred_element_type=F32)

    @pl.when(i == 0)
    def _():
        s0_ref[...] = jnp.zeros_like(s0_ref)
        s1_ref[...] = jnp.zeros_like(s1_ref)
        cnt_ref[...] = jnp.zeros_like(cnt_ref)

    s0_ref[...] += s0
    s1_ref[...] += s1
    cnt_ref[...] += cnt


def _pool_call(ta, tb, ga, gb, dinv, b3a, b3b, batchp):
    return pl.pallas_call(
        _pool_body,
        grid=(GRID,),
        in_specs=[_prow()] * 5 + [_full_spec(1, 128)] * 2
        + [pl.BlockSpec((PB, 4), lambda i: (i, 0))],
        out_specs=(_full_spec(NG, HH), _full_spec(NG, HH), _full_spec(NG, 1)),
        out_shape=(jax.ShapeDtypeStruct((NG, HH), F32),
                   jax.ShapeDtypeStruct((NG, HH), F32),
                   jax.ShapeDtypeStruct((NG, 1), F32)),
    )(ta, tb, ga, gb, dinv, b3a, b3b, batchp)


def _head_body(s0, s1, cnt, wf1, bf1, wf2, bf2, out_ref):
    denom = jnp.maximum(cnt[...], 1.0)
    p0 = s0[...] / denom
    p1 = s1[...] / denom
    h = jnp.maximum(
        _mm(p0, wf1[...][0:HH, :]) + _mm(p1, wf1[...][HH:H, :]) + bf1[...],
        0.0)
    logits = _mm(h, wf2[...]) + bf2[...]
    m = jnp.max(logits, axis=1, keepdims=True)
    e = jnp.exp(logits - m)
    lse = jnp.log(jnp.sum(e, axis=1, keepdims=True)) + m
    out_ref[...] = logits - lse


def _head_call(s0, s1, cnt, wf1, bf1, wf2, bf2):
    return pl.pallas_call(
        _head_body,
        out_shape=jax.ShapeDtypeStruct((NG, 10), F32),
    )(s0, s1, cnt, wf1, bf1, wf2, bf2)


def _bd4(w):
    return jnp.kron(jnp.eye(4, dtype=F32), w)


def _tile4(b):
    return jnp.tile(b, 4).reshape(1, 128)


def kernel(x, edge_index, batch, W1, b1, W2, b2, W3, b3, Wf1, bf1, Wf2, bf2):
    dst = edge_index[1]
    src = lax.optimization_barrier(edge_index)[0]
    x4 = jnp.pad(x, ((0, NPAD - N), (0, 0))).reshape(PK, 8)
    sel = jnp.zeros((8, 128), F32)
    for j in range(4):
        sel = sel.at[2 * j, HH * j].set(1.0).at[2 * j + 1, HH * j + 1].set(1.0)
    batchp = jnp.pad(batch, (0, NPAD - N), constant_values=-1).reshape(PK, 4)

    w1pad = jnp.zeros((HH, H), F32).at[0:2, :].set(W1)
    w1a, w1b = _bd4(w1pad[:, :HH]), _bd4(w1pad[:, HH:])
    b1a, b1b = _tile4(b1[:HH]), _tile4(b1[HH:])
    w2aa, w2ab = _bd4(W2[:HH, :HH]), _bd4(W2[:HH, HH:])
    w2ba, w2bb = _bd4(W2[HH:, :HH]), _bd4(W2[HH:, HH:])
    b2a, b2b = _tile4(b2[:HH]), _tile4(b2[HH:])
    w3aa, w3ab = _bd4(W3[:HH, :HH]), _bd4(W3[:HH, HH:])
    w3ba, w3bb = _bd4(W3[HH:, :HH]), _bd4(W3[HH:, HH:])
    b3a, b3b = _tile4(b3[:HH]), _tile4(b3[HH:])

    dega, degb = _deg_call()(dst)
    dinvp, gxp = _prep_call(x4, (dega + degb).reshape(PK, 4), sel)

    src0 = src.reshape(E // AC0, AC0)
    dst0 = dst.reshape(E // AC0, AC0)
    src32 = src.reshape(E // AC32, AC32)
    dst32 = dst.reshape(E // AC32, AC32)
    t0a, t0b = _aggl0_call()(gxp.reshape(NPAD, HH), src0, dst0)
    g1a, g1b = _l1_call(t0a.reshape(PK, 128), t0b.reshape(PK, 128),
                        gxp, dinvp, w1a, w1b, b1a, b1b,
                        w2aa, w2ab, w2ba, w2bb)

    t1a, t1b = _agg32_call()(g1a.reshape(NPAD, HH), g1b.reshape(NPAD, HH),
                             src32, dst32)
    g2a, g2b = _l2_call(t1a.reshape(PK, 128), t1b.reshape(PK, 128),
                        g1a, g1b, dinvp, b2a, b2b,
                        w3aa, w3ab, w3ba, w3bb)

    t2a, t2b = _agg32_call()(g2a.reshape(NPAD, HH), g2b.reshape(NPAD, HH),
                             src32, dst32)
    s0, s1, cnt = _pool_call(t2a.reshape(PK, 128), t2b.reshape(PK, 128),
                             g2a, g2b, dinvp, b3a, b3b, batchp)

    return _head_call(s0, s1, cnt, Wf1, bf1.reshape(1, HH), Wf2,
                      bf2.reshape(1, 10))


# speedup vs baseline: 1.0471x; 1.0471x over previous
import functools

import jax
import jax.numpy as jnp
from jax import lax
from jax.experimental import pallas as pl
from jax.experimental.pallas import tpu as pltpu
from jax.experimental.pallas import tpu_sc as plsc

N = 50000
E = 800000
NG = 512
H = 64
HH = 32

NC = 2
NS = 16
SLAB = 3136
NPAD = NS * SLAB
BN = SLAB
GRID = NPAD // BN
PK = NPAD * HH // 128
PB = PK // GRID

F32 = jnp.float32


@functools.cache
def _mesh():
    return plsc.VectorSubcoreMesh(core_axis_name="c", subcore_axis_name="s",
                                  num_cores=NC, num_subcores=NS)


EC = E // (NC * NS)
DC = 1000
AC0 = 200
AC32 = 400
EPS = E // NS
SG = 5


def _fill_f32(ref, n, val):
    def body(i, _):
        ref[pl.ds(i * 16, 16)] = jnp.full((16,), val, F32)
        return 0
    lax.fori_loop(0, n // 16, body, 0)


def _fill_rows32(ref, nrows):
    def body(i, _):
        ref[i, pl.ds(0, 16)] = jnp.zeros((16,), F32)
        ref[i, pl.ds(16, 16)] = jnp.zeros((16,), F32)
        return 0
    lax.fori_loop(0, nrows, body, 0)


def _deg_body(dst_hbm, out0, out1, dst_v, ones_v, zrow_v, acc, sem):
    c = lax.axis_index("c")
    s = lax.axis_index("s")
    _fill_f32(zrow_v, SLAB, 0.0)
    _fill_f32(ones_v, DC, 1.0)
    pltpu.sync_copy(zrow_v, acc.at[pl.ds(s * SLAB, SLAB)])
    plsc.subcore_barrier()
    wid = s * NC + c
    def chunk(k, _):
        base = wid * EC + k * DC
        pltpu.sync_copy(dst_hbm.at[pl.ds(base, DC)], dst_v)
        pltpu.async_copy(ones_v, acc.at[dst_v], sem, add=True).wait()
        return 0
    lax.fori_loop(0, EC // DC, chunk, 0)
    plsc.subcore_barrier()
    sl = pl.ds(s * SLAB, SLAB)
    pltpu.sync_copy(acc.at[sl], zrow_v)
    @pl.when(c == 0)
    def _():
        pltpu.sync_copy(zrow_v, out0.at[sl])
    @pl.when(c == 1)
    def _():
        pltpu.sync_copy(zrow_v, out1.at[sl])


@functools.cache
def _deg_call():
    return pl.kernel(
        _deg_body,
        out_type=(jax.ShapeDtypeStruct((NPAD,), F32),
                  jax.ShapeDtypeStruct((NPAD,), F32)),
        mesh=_mesh(),
        scratch_types=[
            pltpu.VMEM((DC,), jnp.int32),
            pltpu.VMEM((DC,), F32),
            pltpu.VMEM((SLAB,), F32),
            pltpu.VMEM_SHARED((NPAD,), F32),
            pltpu.SemaphoreType.DMA,
        ],
        compiler_params=pltpu.CompilerParams(use_tc_tiling_on_sc=False),
    )


def _agg_pipeline(tab_hbm, src2_hbm, dst2_hbm, acc, row0, nch, ac,
                  srcb, dstb, rows, sems):
    nsg = nch // SG
    pltpu.sync_copy(src2_hbm.at[pl.ds(row0, SG)], srcb)
    pltpu.sync_copy(dst2_hbm.at[pl.ds(row0, SG)], dstb)
    pltpu.async_copy(tab_hbm.at[srcb.at[0]], rows[0], sems[0])

    def group(sg, _):
        for j in range(SG - 1):
            p = j % 2
            pltpu.async_copy(tab_hbm.at[srcb.at[j + 1]], rows[1 - p],
                             sems[1 - p])
            pltpu.make_async_copy(tab_hbm.at[srcb.at[j]], rows[p],
                                  sems[p]).wait()
            pltpu.sync_copy(rows[p], acc.at[dstb.at[j]], add=True)
        p = (SG - 1) % 2
        pltpu.make_async_copy(tab_hbm.at[srcb.at[SG - 1]], rows[p],
                              sems[p]).wait()
        pltpu.sync_copy(rows[p], acc.at[dstb.at[SG - 1]], add=True)

        @pl.when(sg < nsg - 1)
        def _():
            nrow = row0 + (sg + 1) * SG
            pltpu.sync_copy(src2_hbm.at[pl.ds(nrow, SG)], srcb)
            pltpu.sync_copy(dst2_hbm.at[pl.ds(nrow, SG)], dstb)
            pltpu.async_copy(tab_hbm.at[srcb.at[0]], rows[0], sems[0])
        return 0

    lax.fori_loop(0, nsg, group, 0)


def _agg_scratch(ac):
    return (
        [pltpu.VMEM((SG, ac), jnp.int32)] * 2
        + [pltpu.VMEM((ac, HH), F32)] * 2
        + [pltpu.VMEM_SHARED((NPAD, HH), F32)]
        + [pltpu.SemaphoreType.DMA] * 2
    )


def _stg_rows(ac):
    return 392 if ac >= 392 else 196


def _zero_acc(acc, s, rows0, stgn):
    _fill_rows32(rows0, stgn)
    stg = rows0.at[pl.ds(0, stgn)]
    for j in range(SLAB // stgn):
        pltpu.sync_copy(stg, acc.at[pl.ds(s * SLAB + j * stgn, stgn)])


def _acc_to_out(acc, s, rows0, out, stgn):
    stg = rows0.at[pl.ds(0, stgn)]
    for j in range(SLAB // stgn):
        sl = pl.ds(s * SLAB + j * stgn, stgn)
        pltpu.sync_copy(acc.at[sl], stg)
        pltpu.sync_copy(stg, out.at[sl])


def _aggl0_body(tab_hbm, srcd_hbm, dstd_hbm, out0, out1, *sc):
    srcb, dstb, rows, acc, sems = sc[0], sc[1], sc[2:4], sc[4], sc[5:]
    c = lax.axis_index("c")
    s = lax.axis_index("s")
    _zero_acc(acc, s, rows[0], _stg_rows(AC0))
    plsc.subcore_barrier()
    wid = s * NC + c
    _agg_pipeline(tab_hbm, srcd_hbm, dstd_hbm, acc, wid * (EC // AC0),
                  EC // AC0, AC0, srcb, dstb, rows, sems)
    plsc.subcore_barrier()
    @pl.when(c == 0)
    def _():
        _acc_to_out(acc, s, rows[0], out0, _stg_rows(AC0))
    @pl.when(c == 1)
    def _():
        _acc_to_out(acc, s, rows[0], out1, _stg_rows(AC0))


@functools.cache
def _aggl0_call():
    return pl.kernel(
        _aggl0_body,
        out_type=(jax.ShapeDtypeStruct((NPAD, HH), F32),
                  jax.ShapeDtypeStruct((NPAD, HH), F32)),
        mesh=_mesh(),
        scratch_types=_agg_scratch(AC0),
        compiler_params=pltpu.CompilerParams(use_tc_tiling_on_sc=False),
    )


def _agg32_body(taba_hbm, tabb_hbm, srcd_hbm, dstd_hbm, outa, outb, *sc):
    srcb, dstb, rows, acc, sems = sc[0], sc[1], sc[2:4], sc[4], sc[5:]
    c = lax.axis_index("c")
    s = lax.axis_index("s")

    def half(tab_hbm, out):
        _zero_acc(acc, s, rows[0], _stg_rows(AC32))
        plsc.subcore_barrier()
        _agg_pipeline(tab_hbm, srcd_hbm, dstd_hbm, acc, s * (EPS // AC32),
                      EPS // AC32, AC32, srcb, dstb, rows, sems)
        plsc.subcore_barrier()
        _acc_to_out(acc, s, rows[0], out, _stg_rows(AC32))

    @pl.when(c == 0)
    def _():
        half(taba_hbm, outa)
    @pl.when(c == 1)
    def _():
        half(tabb_hbm, outb)


@functools.cache
def _agg32_call():
    return pl.kernel(
        _agg32_body,
        out_type=(jax.ShapeDtypeStruct((NPAD, HH), F32),
                  jax.ShapeDtypeStruct((NPAD, HH), F32)),
        mesh=_mesh(),
        scratch_types=_agg_scratch(AC32),
        compiler_params=pltpu.CompilerParams(use_tc_tiling_on_sc=False),
    )


def _prow(w=128):
    return pl.BlockSpec((PB, w), lambda i: (i, 0))


def _full_spec(a, b):
    return pl.BlockSpec((a, b), lambda i: (0, 0))


def _prep_body(x4_ref, dg_ref, sel_ref, dinv_ref, gx_ref):
    r = lax.rsqrt(dg_ref[...] + 1.0)
    dv = jnp.concatenate(
        [jnp.broadcast_to(r[:, j:j + 1], (PB, HH)) for j in range(4)], axis=1)
    dinv_ref[...] = dv
    gx_ref[...] = _mm(x4_ref[...], sel_ref[...]) * dv


def _prep_call(x4, dg, sel):
    return pl.pallas_call(
        _prep_body,
        grid=(GRID,),
        in_specs=[_prow(8), _prow(4), _full_spec(8, 128)],
        out_specs=(_prow(), _prow()),
        out_shape=(jax.ShapeDtypeStruct((PK, 128), F32),
                   jax.ShapeDtypeStruct((PK, 128), F32)),
    )(x4, dg, sel)


def _mm(a, b):
    return jnp.dot(a, b, preferred_element_type=F32)


def _l1_body(t0a, t0b, gx, dinv, w1a, w1b, b1a, b1b,
             w2aa, w2ab, w2ba, w2bb, ga_ref, gb_ref):
    d = dinv[...]
    q = (t0a[...] + t0b[...] + gx[...]) * d
    y1a = jnp.maximum(_mm(q, w1a[...]) + b1a[...], 0.0)
    y1b = jnp.maximum(_mm(q, w1b[...]) + b1b[...], 0.0)
    ga_ref[...] = (_mm(y1a, w2aa[...]) + _mm(y1b, w2ba[...])) * d
    gb_ref[...] = (_mm(y1a, w2ab[...]) + _mm(y1b, w2bb[...])) * d


def _l1_call(t0a, t0b, gx, dinv, w1a, w1b, b1a, b1b, w2aa, w2ab, w2ba, w2bb):
    return pl.pallas_call(
        _l1_body,
        grid=(GRID,),
        in_specs=[_prow()] * 4 + [_full_spec(128, 128)] * 2
        + [_full_spec(1, 128)] * 2 + [_full_spec(128, 128)] * 4,
        out_specs=(_prow(), _prow()),
        out_shape=(jax.ShapeDtypeStruct((PK, 128), F32),
                   jax.ShapeDtypeStruct((PK, 128), F32)),
    )(t0a, t0b, gx, dinv, w1a, w1b, b1a, b1b, w2aa, w2ab, w2ba, w2bb)


def _l2_body(ta, tb, ga, gb, dinv, b2a, b2b,
             w3aa, w3ab, w3ba, w3bb, oa_ref, ob_ref):
    d = dinv[...]
    u0 = jnp.maximum((ta[...] + ga[...]) * d + b2a[...], 0.0)
    u1 = jnp.maximum((tb[...] + gb[...]) * d + b2b[...], 0.0)
    oa_ref[...] = (_mm(u0, w3aa[...]) + _mm(u1, w3ba[...])) * d
    ob_ref[...] = (_mm(u0, w3ab[...]) + _mm(u1, w3bb[...])) * d


def _l2_call(ta, tb, ga, gb, dinv, b2a, b2b, w3aa, w3ab, w3ba, w3bb):
    return pl.pallas_call(
        _l2_body,
        grid=(GRID,),
        in_specs=[_prow()] * 5 + [_full_spec(1, 128)] * 2
        + [_full_spec(128, 128)] * 4,
        out_specs=(_prow(), _prow()),
        out_shape=(jax.ShapeDtypeStruct((PK, 128), F32),
                   jax.ShapeDtypeStruct((PK, 128), F32)),
    )(ta, tb, ga, gb, dinv, b2a, b2b, w3aa, w3ab, w3ba, w3bb)


def _pool_body(ta, tb, ga, gb, dinv, b3a, b3b, batch,
               s0_ref, s1_ref, cnt_ref):
    i = pl.program_id(0)
    d = dinv[...]
    v0 = jnp.maximum((ta[...] + ga[...]) * d + b3a[...], 0.0)
    v1 = jnp.maximum((tb[...] + gb[...]) * d + b3b[...], 0.0)
    ids = batch[...]
    seg = lax.broadcasted_iota(jnp.int32, (PB, NG), 1)
    dn = (((0,), (0,)), ((), ()))
    v0h = v0.astype(jnp.bfloat16)
    v1h = v1.astype(jnp.bfloat16)
    ones = jnp.ones((PB, 1), jnp.bfloat16)
    s0 = jnp.zeros((NG, HH), F32)
    s1 = jnp.zeros((NG, HH), F32)
    cnt = jnp.zeros((NG, 1), F32)
    for j in range(4):
        p = (ids[:, j:j + 1] == seg).astype(jnp.bfloat16)
        s0 = s0 + lax.dot_general(p, v0h[:, HH * j:HH * (j + 1)], dn,
                                  preferred_element_type=F32)
        s1 = s1 + lax.dot_general(p, v1h[:, HH * j:HH * (j + 1)], dn,
                                  preferred_element_type=F32)
        cnt = cnt + lax.dot_general(p, ones, dn,
                                    preferred_element_type=F32)

    @pl.when(i == 0)
    def _():
        s0_ref[...] = jnp.zeros_like(s0_ref)
        s1_ref[...] = jnp.zeros_like(s1_ref)
        cnt_ref[...] = jnp.zeros_like(cnt_ref)

    s0_ref[...] += s0
    s1_ref[...] += s1
    cnt_ref[...] += cnt


def _pool_call(ta, tb, ga, gb, dinv, b3a, b3b, batchp):
    return pl.pallas_call(
        _pool_body,
        grid=(GRID,),
        in_specs=[_prow()] * 5 + [_full_spec(1, 128)] * 2
        + [pl.BlockSpec((PB, 4), lambda i: (i, 0))],
        out_specs=(_full_spec(NG, HH), _full_spec(NG, HH), _full_spec(NG, 1)),
        out_shape=(jax.ShapeDtypeStruct((NG, HH), F32),
                   jax.ShapeDtypeStruct((NG, HH), F32),
                   jax.ShapeDtypeStruct((NG, 1), F32)),
    )(ta, tb, ga, gb, dinv, b3a, b3b, batchp)


def _head_body(s0, s1, cnt, wf1, bf1, wf2, bf2, out_ref):
    denom = jnp.maximum(cnt[...], 1.0)
    p0 = s0[...] / denom
    p1 = s1[...] / denom
    h = jnp.maximum(
        _mm(p0, wf1[...][0:HH, :]) + _mm(p1, wf1[...][HH:H, :]) + bf1[...],
        0.0)
    logits = _mm(h, wf2[...]) + bf2[...]
    m = jnp.max(logits, axis=1, keepdims=True)
    e = jnp.exp(logits - m)
    lse = jnp.log(jnp.sum(e, axis=1, keepdims=True)) + m
    out_ref[...] = logits - lse


def _head_call(s0, s1, cnt, wf1, bf1, wf2, bf2):
    return pl.pallas_call(
        _head_body,
        out_shape=jax.ShapeDtypeStruct((NG, 10), F32),
    )(s0, s1, cnt, wf1, bf1, wf2, bf2)


def _bd4(w):
    return jnp.kron(jnp.eye(4, dtype=F32), w)


def _tile4(b):
    return jnp.tile(b, 4).reshape(1, 128)


def kernel(x, edge_index, batch, W1, b1, W2, b2, W3, b3, Wf1, bf1, Wf2, bf2):
    src = edge_index[0]
    dst = edge_index[1]
    x4 = jnp.pad(x, ((0, NPAD - N), (0, 0))).reshape(PK, 8)
    sel = jnp.zeros((8, 128), F32)
    for j in range(4):
        sel = sel.at[2 * j, HH * j].set(1.0).at[2 * j + 1, HH * j + 1].set(1.0)
    batchp = jnp.pad(batch, (0, NPAD - N), constant_values=-1).reshape(PK, 4)

    w1pad = jnp.zeros((HH, H), F32).at[0:2, :].set(W1)
    w1a, w1b = _bd4(w1pad[:, :HH]), _bd4(w1pad[:, HH:])
    b1a, b1b = _tile4(b1[:HH]), _tile4(b1[HH:])
    w2aa, w2ab = _bd4(W2[:HH, :HH]), _bd4(W2[:HH, HH:])
    w2ba, w2bb = _bd4(W2[HH:, :HH]), _bd4(W2[HH:, HH:])
    b2a, b2b = _tile4(b2[:HH]), _tile4(b2[HH:])
    w3aa, w3ab = _bd4(W3[:HH, :HH]), _bd4(W3[:HH, HH:])
    w3ba, w3bb = _bd4(W3[HH:, :HH]), _bd4(W3[HH:, HH:])
    b3a, b3b = _tile4(b3[:HH]), _tile4(b3[HH:])

    dega, degb = _deg_call()(dst)
    dinvp, gxp = _prep_call(x4, (dega + degb).reshape(PK, 4), sel)

    src0 = src.reshape(E // AC0, AC0)
    dst0 = dst.reshape(E // AC0, AC0)
    src32 = src.reshape(E // AC32, AC32)
    dst32 = dst.reshape(E // AC32, AC32)
    t0a, t0b = _aggl0_call()(gxp.reshape(NPAD, HH), src0, dst0)
    g1a, g1b = _l1_call(t0a.reshape(PK, 128), t0b.reshape(PK, 128),
                        gxp, dinvp, w1a, w1b, b1a, b1b,
                        w2aa, w2ab, w2ba, w2bb)

    t1a, t1b = _agg32_call()(g1a.reshape(NPAD, HH), g1b.reshape(NPAD, HH),
                             src32, dst32)
    g2a, g2b = _l2_call(t1a.reshape(PK, 128), t1b.reshape(PK, 128),
                        g1a, g1b, dinvp, b2a, b2b,
                        w3aa, w3ab, w3ba, w3bb)

    t2a, t2b = _agg32_call()(g2a.reshape(NPAD, HH), g2b.reshape(NPAD, HH),
                             src32, dst32)
    s0, s1, cnt = _pool_call(t2a.reshape(PK, 128), t2b.reshape(PK, 128),
                             g2a, g2b, dinvp, b3a, b3b, batchp)

    return _head_call(s0, s1, cnt, Wf1, bf1.reshape(1, HH), Wf2,
                      bf2.reshape(1, 10))
